# baseline (speedup 1.0000x reference)
import dataclasses
import functools

import jax
import jax.numpy as jnp
from jax import lax
from jax.experimental import pallas as pl
from jax.experimental.pallas import tpu as pltpu
from jax.experimental.pallas import tpu_sc as plsc

B = 16384
F = 64
L = 16
NC = 2
NS = 16
NW = NC * NS
BPW = B // NW
GCH = 128
NCHUNK = BPW // GCH
GROUPS = BPW // L


def _mf_body(ue_hbm, ub_hbm, ie_hbm, ib_hbm, uid_hbm, iid_hbm, out_hbm,
             uidx, iidx, uhi, ihi, u_rows, i_rows, ubr, ibr, out_v, sem):
    wid = lax.axis_index("s") * NC + lax.axis_index("c")
    base = wid * BPW

    cps = []
    for k in range(NCHUNK):
        cps.append(pltpu.async_copy(
            uid_hbm.at[pl.ds(base + k * GCH, GCH)], uidx.at[k], sem))
        cps.append(pltpu.async_copy(
            iid_hbm.at[pl.ds(base + k * GCH, GCH)], iidx.at[k], sem))
    for c in cps:
        c.wait()

    for k in range(NCHUNK):
        for j in range(GCH // L):
            sl = pl.ds(j * L, L)
            uhi[k, sl] = uidx[k, sl] >> 4
            ihi[k, sl] = iidx[k, sl] >> 4

    cps = []
    for k in range(NCHUNK):
        sl = pl.ds(k * GCH, GCH)
        cps.append(pltpu.async_copy(ue_hbm.at[uidx.at[k]], u_rows.at[sl], sem))
        cps.append(pltpu.async_copy(ie_hbm.at[iidx.at[k]], i_rows.at[sl], sem))
        cps.append(pltpu.async_copy(ub_hbm.at[uhi.at[k]], ubr.at[sl], sem))
        cps.append(pltpu.async_copy(ib_hbm.at[ihi.at[k]], ibr.at[sl], sem))
    for c in cps:
        c.wait()

    lane = lax.iota(jnp.int32, L)

    @pl.loop(0, GROUPS)
    def _(g):
        rows = g * L + lane
        uid_v = plsc.load_gather(uidx, [rows >> 7, rows & 127])
        iid_v = plsc.load_gather(iidx, [rows >> 7, rows & 127])
        acc = (plsc.load_gather(ubr, [rows, uid_v & 15])
               + plsc.load_gather(ibr, [rows, iid_v & 15]))
        for f in range(F):
            col = jnp.full((L,), f, jnp.int32)
            uv = plsc.load_gather(u_rows, [rows, col])
            iv = plsc.load_gather(i_rows, [rows, col])
            acc = acc + uv * iv
        out_v[pl.ds(g * L, L)] = 1.0 / (1.0 + jnp.exp(-acc))

    pltpu.sync_copy(out_v, out_hbm.at[pl.ds(base, BPW)])


@jax.jit
def _mf(user_embed, user_bias_embed, item_embed, item_bias_embed,
        user_ids, item_ids):
    cp = pltpu.CompilerParams()
    fields = pltpu.CompilerParams.__dataclass_fields__
    if "needs_layout_passes" in fields:
        cp = dataclasses.replace(cp, needs_layout_passes=False)
    if "use_tc_tiling_on_sc" in fields:
        cp = dataclasses.replace(cp, use_tc_tiling_on_sc=False)
    run = pl.kernel(
        _mf_body,
        out_type=jax.ShapeDtypeStruct((B,), jnp.float32),
        compiler_params=cp,
        mesh=plsc.VectorSubcoreMesh(core_axis_name="c", subcore_axis_name="s"),
        scratch_types=[
            pltpu.VMEM((NCHUNK, GCH), jnp.int32),
            pltpu.VMEM((NCHUNK, GCH), jnp.int32),
            pltpu.VMEM((NCHUNK, GCH), jnp.int32),
            pltpu.VMEM((NCHUNK, GCH), jnp.int32),
            pltpu.VMEM((BPW, F), jnp.float32),
            pltpu.VMEM((BPW, F), jnp.float32),
            pltpu.VMEM((BPW, L), jnp.float32),
            pltpu.VMEM((BPW, L), jnp.float32),
            pltpu.VMEM((BPW,), jnp.float32),
            pltpu.SemaphoreType.DMA,
        ],
    )
    return run(user_embed, user_bias_embed.reshape(-1, L),
               item_embed, item_bias_embed.reshape(-1, L),
               user_ids, item_ids)


def kernel(user_embed, user_bias_embed, item_embed, item_bias_embed,
           user_ids, item_ids):
    return _mf(user_embed, user_bias_embed, item_embed, item_bias_embed,
               user_ids.astype(jnp.int32), item_ids.astype(jnp.int32))


# speedup vs baseline: 1.0009x; 1.0009x over previous
import dataclasses
import functools

import jax
import jax.numpy as jnp
from jax import lax
from jax.experimental import pallas as pl
from jax.experimental.pallas import tpu as pltpu
from jax.experimental.pallas import tpu_sc as plsc

B = 16384
F = 64
L = 16
NC = 2
NS = 16
NW = NC * NS
BPW = B // NW
GCH = 128
NCHUNK = BPW // GCH
GROUPS = BPW // L


def _mf_body(ue_hbm, ubT_hbm, ie_hbm, ibT_hbm, uid_hbm, iid_hbm, out_hbm,
             uidx, iidx, u_rows, i_rows, ubv, ibv, out_v, sem):
    wid = lax.axis_index("s") * NC + lax.axis_index("c")
    base = wid * BPW

    cps = []
    for k in range(NCHUNK):
        cps.append(pltpu.async_copy(
            uid_hbm.at[pl.ds(base + k * GCH, GCH)], uidx.at[k], sem))
        cps.append(pltpu.async_copy(
            iid_hbm.at[pl.ds(base + k * GCH, GCH)], iidx.at[k], sem))
    for c in cps:
        c.wait()

    cps = []
    for k in range(NCHUNK):
        sl = pl.ds(k * GCH, GCH)
        cps.append(pltpu.async_copy(ue_hbm.at[uidx.at[k]], u_rows.at[sl], sem))
        cps.append(pltpu.async_copy(ie_hbm.at[iidx.at[k]], i_rows.at[sl], sem))
        cps.append(pltpu.async_copy(ubT_hbm.at[0].at[uidx.at[k]], ubv.at[sl], sem))
        cps.append(pltpu.async_copy(ibT_hbm.at[0].at[iidx.at[k]], ibv.at[sl], sem))
    for c in cps:
        c.wait()

    lane = lax.iota(jnp.int32, L)

    @pl.loop(0, GROUPS)
    def _(g):
        sl16 = pl.ds(g * L, L)
        rows = g * L + lane
        acc = ubv[sl16] + ibv[sl16]
        for f in range(F):
            col = jnp.full((L,), f, jnp.int32)
            uv = plsc.load_gather(u_rows, [rows, col])
            iv = plsc.load_gather(i_rows, [rows, col])
            acc = acc + uv * iv
        out_v[sl16] = 1.0 / (1.0 + jnp.exp(-acc))

    pltpu.sync_copy(out_v, out_hbm.at[pl.ds(base, BPW)])


@jax.jit
def _mf(user_embed, user_bias_embed, item_embed, item_bias_embed,
        user_ids, item_ids):
    cp = pltpu.CompilerParams()
    fields = pltpu.CompilerParams.__dataclass_fields__
    if "needs_layout_passes" in fields:
        cp = dataclasses.replace(cp, needs_layout_passes=False)
    if "use_tc_tiling_on_sc" in fields:
        cp = dataclasses.replace(cp, use_tc_tiling_on_sc=False)
    run = pl.kernel(
        _mf_body,
        out_type=jax.ShapeDtypeStruct((B,), jnp.float32),
        compiler_params=cp,
        mesh=plsc.VectorSubcoreMesh(core_axis_name="c", subcore_axis_name="s"),
        scratch_types=[
            pltpu.VMEM((NCHUNK, GCH), jnp.int32),
            pltpu.VMEM((NCHUNK, GCH), jnp.int32),
            pltpu.VMEM((BPW, F), jnp.float32),
            pltpu.VMEM((BPW, F), jnp.float32),
            pltpu.VMEM((BPW,), jnp.float32),
            pltpu.VMEM((BPW,), jnp.float32),
            pltpu.VMEM((BPW,), jnp.float32),
            pltpu.SemaphoreType.DMA,
        ],
    )
    return run(user_embed, user_bias_embed.T, item_embed, item_bias_embed.T,
               user_ids, item_ids)


def kernel(user_embed, user_bias_embed, item_embed, item_bias_embed,
           user_ids, item_ids):
    return _mf(user_embed, user_bias_embed, item_embed, item_bias_embed,
               user_ids.astype(jnp.int32), item_ids.astype(jnp.int32))
